# baseline (speedup 1.0000x reference)
import jax
import jax.numpy as jnp
from jax.experimental import pallas as pl

_EPS = 1e-6


def _pass1_kernel(x_ref, z_ref, h_ref, w_ref,
                  wpsi_ref, bpsi_ref, wphi_ref, bphi_ref,
                  wg1_ref, bg1_ref, wg2_ref, bg2_ref,
                  gate_ref, e1_ref, de_ref, dv_ref):
    i = pl.program_id(0)
    half = wpsi_ref.shape[1]
    px = jnp.dot(x_ref[:], wpsi_ref[:], preferred_element_type=jnp.float32) + bpsi_ref[:]
    pz = jnp.dot(z_ref[:], wphi_ref[:], preferred_element_type=jnp.float32) + bphi_ref[:]
    g1 = jax.nn.relu(
        jnp.dot(px, wg1_ref[0:half, :], preferred_element_type=jnp.float32)
        + jnp.dot(pz, wg1_ref[half:, :], preferred_element_type=jnp.float32)
        + bg1_ref[:])
    gate = jax.nn.sigmoid(
        jnp.dot(g1, wg2_ref[:], preferred_element_type=jnp.float32) + bg2_ref[:])
    fused = gate * pz + (1.0 - gate) * px
    gate_ref[:] = gate

    h = h_ref[:]
    dv_ref[:] = jnp.clip(jnp.sum(h * w_ref[:], axis=1, keepdims=True), _EPS, None)

    @pl.when(i == 0)
    def _():
        e1_ref[:] = jnp.zeros_like(e1_ref)
        de_ref[:] = jnp.zeros_like(de_ref)

    de_ref[:] += jnp.sum(h, axis=0, keepdims=True)
    e1_ref[:] += jax.lax.dot_general(fused, h, (((0,), (0,)), ((), ())),
                                     preferred_element_type=jnp.float32)


def _pass2_kernel(h_ref, e1_ref, de_ref, dv_ref, w_ref, w1_ref, b1_ref, e2_ref):
    i = pl.program_id(0)
    s = w_ref[:] / jnp.clip(de_ref[:], _EPS, None)
    h = h_ref[:]
    e1 = jnp.transpose(e1_ref[:])
    agg = jnp.dot(h * s, e1, preferred_element_type=jnp.float32) / dv_ref[:]
    hid = jax.nn.relu(
        jnp.dot(agg, w1_ref[:], preferred_element_type=jnp.float32) + b1_ref[:])

    @pl.when(i == 0)
    def _():
        e2_ref[:] = jnp.zeros_like(e2_ref)

    e2_ref[:] += jax.lax.dot_general(hid, h, (((0,), (0,)), ((), ())),
                                     preferred_element_type=jnp.float32)


def _pass3_kernel(h_ref, e2_ref, de_ref, dv_ref, w_ref, w2_ref, b2_ref,
                  wo_ref, bo_ref, out_ref):
    s = w_ref[:] / jnp.clip(de_ref[:], _EPS, None)
    e2 = jnp.transpose(e2_ref[:])
    agg = jnp.dot(h_ref[:] * s, e2, preferred_element_type=jnp.float32) / dv_ref[:]
    o = jax.nn.relu(
        jnp.dot(agg, w2_ref[:], preferred_element_type=jnp.float32) + b2_ref[:])
    out_ref[:] = jnp.dot(o, wo_ref[:], preferred_element_type=jnp.float32) + bo_ref[:]


def _pick_tile(n):
    for t in (400, 250, 200, 128, 125, 100, 80, 64, 50, 40,
              32, 25, 20, 16, 10, 8, 5, 4, 2):
        if n % t == 0:
            return t
    return n


def kernel(x, z, incidence, edge_weights, Wpsi, bpsi, Wphi, bphi,
           Wg1, bg1, Wg2, bg2, W1, b1, W2, b2, Wo, bo):
    n, m = incidence.shape
    half = Wpsi.shape[1]
    hid = W1.shape[1]
    out_dim = Wo.shape[1]
    tn = _pick_tile(n)
    grid = (n // tn,)

    w2d = edge_weights.reshape(1, m)

    def row(b):
        return b.reshape(1, -1)

    def full(shape):
        return pl.BlockSpec(shape, lambda i: (0,) * len(shape))

    def tile(r, c):
        return pl.BlockSpec((r, c), lambda i: (i, 0))

    f32 = jnp.float32

    gate, e1, de, dv = pl.pallas_call(
        _pass1_kernel,
        grid=grid,
        in_specs=[tile(tn, x.shape[1]), tile(tn, z.shape[1]), tile(tn, m),
                  full((1, m)),
                  full(Wpsi.shape), full((1, half)),
                  full(Wphi.shape), full((1, half)),
                  full(Wg1.shape), full((1, Wg1.shape[1])),
                  full(Wg2.shape), full((1, half))],
        out_specs=[tile(tn, half), full((half, m)), full((1, m)), tile(tn, 1)],
        out_shape=[jax.ShapeDtypeStruct((n, half), f32),
                   jax.ShapeDtypeStruct((half, m), f32),
                   jax.ShapeDtypeStruct((1, m), f32),
                   jax.ShapeDtypeStruct((n, 1), f32)],
    )(x, z, incidence, w2d, Wpsi, row(bpsi), Wphi, row(bphi),
      Wg1, row(bg1), Wg2, row(bg2))

    e2 = pl.pallas_call(
        _pass2_kernel,
        grid=grid,
        in_specs=[tile(tn, m), full((half, m)), full((1, m)), tile(tn, 1),
                  full((1, m)), full(W1.shape), full((1, hid))],
        out_specs=full((hid, m)),
        out_shape=jax.ShapeDtypeStruct((hid, m), f32),
    )(incidence, e1, de, dv, w2d, W1, row(b1))

    logits = pl.pallas_call(
        _pass3_kernel,
        grid=grid,
        in_specs=[tile(tn, m), full((hid, m)), full((1, m)), tile(tn, 1),
                  full((1, m)), full(W2.shape), full((1, hid)),
                  full(Wo.shape), full((1, out_dim))],
        out_specs=tile(tn, out_dim),
        out_shape=jax.ShapeDtypeStruct((n, out_dim), f32),
    )(incidence, e2, de, dv, w2d, W2, row(b2), Wo, row(bo))

    return (logits, gate)
